# baseline (speedup 1.0000x reference)
import jax
import jax.numpy as jnp
from jax.experimental import pallas as pl
from jax.experimental.pallas import tpu as pltpu


def _scale_block(x_ref, d_ref, o_ref):
    o_ref[...] = x_ref[...] * d_ref[...]


def kernel(x, diag):
    D, C = x.shape
    BR = 8192
    diag2 = diag.reshape(D, 1)
    return pl.pallas_call(
        _scale_block,
        grid=(D // BR,),
        in_specs=[
            pl.BlockSpec((BR, C), lambda i: (i, 0)),
            pl.BlockSpec((BR, 1), lambda i: (i, 0)),
        ],
        out_specs=pl.BlockSpec((BR, C), lambda i: (i, 0)),
        out_shape=jax.ShapeDtypeStruct((D, C), x.dtype),
        compiler_params=pltpu.CompilerParams(
            dimension_semantics=("parallel",),
        ),
    )(x, diag2)


# speedup vs baseline: 7.8313x; 7.8313x over previous
import jax
import jax.numpy as jnp
from jax.experimental import pallas as pl
from jax.experimental.pallas import tpu as pltpu


def _scale_block(x_ref, d_ref, o_ref):
    o_ref[...] = x_ref[...] * d_ref[...]


def kernel(x, diag):
    D, C = x.shape
    xt = x.T
    dv = diag.reshape(1, D)
    BL = 8192
    out_t = pl.pallas_call(
        _scale_block,
        grid=(D // BL,),
        in_specs=[
            pl.BlockSpec((C, BL), lambda i: (0, i)),
            pl.BlockSpec((1, BL), lambda i: (0, i)),
        ],
        out_specs=pl.BlockSpec((C, BL), lambda i: (0, i)),
        out_shape=jax.ShapeDtypeStruct((C, D), x.dtype),
        compiler_params=pltpu.CompilerParams(
            dimension_semantics=("parallel",),
        ),
    )(xt, dv)
    return out_t.T


# speedup vs baseline: 8.4874x; 1.0838x over previous
import jax
import jax.numpy as jnp
from jax.experimental import pallas as pl
from jax.experimental.pallas import tpu as pltpu


def _scale_block(x_ref, d_ref, o_ref):
    o_ref[...] = x_ref[...] * d_ref[...]


def kernel(x, diag):
    D, C = x.shape
    xt = x.T
    dv = diag.reshape(1, D)
    BL = 32768
    out_t = pl.pallas_call(
        _scale_block,
        grid=(D // BL,),
        in_specs=[
            pl.BlockSpec((C, BL), lambda i: (0, i)),
            pl.BlockSpec((1, BL), lambda i: (0, i)),
        ],
        out_specs=pl.BlockSpec((C, BL), lambda i: (0, i)),
        out_shape=jax.ShapeDtypeStruct((C, D), x.dtype),
        compiler_params=pltpu.CompilerParams(
            dimension_semantics=("parallel",),
        ),
    )(xt, dv)
    return out_t.T
